# baseline (speedup 1.0000x reference)
import functools

import jax
import jax.numpy as jnp
from jax import lax
from jax.experimental import pallas as pl
from jax.experimental.pallas import tpu as pltpu
from jax.experimental.pallas import tpu_sc as plsc

SEQ = 200
EMB = 32
NC = 2
NS = 16
NW = NC * NS
LANES = 16

VOCAB = 1000000
TILE_W = 128
N_FULL_TILES = VOCAB // TILE_W
TAIL = VOCAB - N_FULL_TILES * TILE_W

CHUNK_SEQS = 8
CHUNK = CHUNK_SEQS * SEQ
N_CHUNKS = 4


BLK_TILES = 4
BLK_W = BLK_TILES * TILE_W
N_BLOCKS = N_FULL_TILES // BLK_TILES
SKEW = EMB + 1


def _detile_table(table_t, tail_flat):
    mesh = plsc.VectorSubcoreMesh(
        core_axis_name="c", subcore_axis_name="s", num_cores=NC, num_subcores=NS
    )

    @functools.partial(
        pl.kernel,
        mesh=mesh,
        compiler_params=pltpu.CompilerParams(
            use_tc_tiling_on_sc=True, needs_layout_passes=False
        ),
        out_type=jax.ShapeDtypeStruct((VOCAB * EMB,), jnp.float32),
        scratch_types=[
            pltpu.VMEM((EMB, BLK_W), jnp.float32),
            pltpu.VMEM((EMB, BLK_W), jnp.float32),
            pltpu.VMEM((BLK_W * SKEW,), jnp.float32),
            pltpu.VMEM((BLK_W * EMB,), jnp.float32),
            pltpu.VMEM((BLK_W * EMB,), jnp.float32),
            pltpu.VMEM((TAIL * EMB,), jnp.float32),
            pltpu.SemaphoreType.DMA,
            pltpu.SemaphoreType.DMA,
            pltpu.SemaphoreType.DMA,
            pltpu.SemaphoreType.DMA,
        ],
    )
    def body(tt_hbm, tail_hbm, out_hbm, in0, in1, skew_v, st0, st1, tailv,
             gi0, gi1, wo0, wo1):
        wid = lax.axis_index("s") * NC + lax.axis_index("c")
        ins = (in0, in1)
        sts = (st0, st1)
        gsems = (gi0, gi1)
        wsems = (wo0, wo1)

        n_mine = N_BLOCKS // NW + jnp.where(wid < N_BLOCKS % NW, 1, 0)

        lane = lax.iota(jnp.int32, LANES)
        scat_base = lane * SKEW

        def fetch(k, b):
            v0 = (wid + k * NW) * BLK_W
            for t in range(EMB // 8):
                pltpu.async_copy(
                    tt_hbm.at[pl.ds(8 * t, 8), pl.ds(v0, BLK_W)],
                    ins[b].at[pl.ds(8 * t, 8), :],
                    gsems[b],
                )

        def transpose(b):
            buf = ins[b]
            stg = sts[b]

            @plsc.parallel_loop(0, EMB, unroll=4)
            def _(d):
                for grp in range(BLK_W // LANES):
                    x = buf[d, pl.ds(grp * LANES, LANES)]
                    idx = scat_base + (grp * LANES * SKEW + d)
                    plsc.store_scatter(skew_v, [idx], x)

            @plsc.parallel_loop(0, BLK_W, unroll=8)
            def _(r):
                stg[pl.ds(r * EMB, LANES)] = skew_v[pl.ds(r * SKEW, LANES)]
                stg[pl.ds(r * EMB + LANES, LANES)] = (
                    skew_v[pl.ds(r * SKEW + LANES, LANES)]
                )

        def write(k, b):
            v0 = (wid + k * NW) * BLK_W
            return pltpu.async_copy(
                sts[b], out_hbm.at[pl.ds(v0 * EMB, BLK_W * EMB)], wsems[b]
            )

        fetch(0, 0)

        def pair_body(kp, carry):
            for b in range(2):
                k = kp * 2 + b

                @pl.when(k < n_mine)
                def _():
                    @pl.when(k + 1 < n_mine)
                    def _():
                        fetch(k + 1, 1 - b)

                    @pl.when(k >= 2)
                    def _():
                        pltpu.make_async_copy(
                            sts[b], out_hbm.at[pl.ds(0, BLK_W * EMB)],
                            wsems[b],
                        ).wait()

                    pltpu.make_async_copy(
                        tt_hbm.at[:, pl.ds(0, BLK_W)], ins[b], gsems[b]
                    ).wait()
                    transpose(b)
                    write(k, b)

            return carry

        lax.fori_loop(0, (n_mine + 1) // 2, pair_body, 0)

        for b in range(2):
            pltpu.make_async_copy(
                sts[b], out_hbm.at[pl.ds(0, BLK_W * EMB)], wsems[b]
            ).wait()

        @pl.when(wid == 0)
        def _():
            pltpu.sync_copy(tail_hbm, tailv)
            pltpu.sync_copy(
                tailv,
                out_hbm.at[pl.ds(N_FULL_TILES * TILE_W * EMB, TAIL * EMB)],
            )

    return body(table_t, tail_flat)


def _gather_add(idx_3d, table_rm, pos_table, n_total):
    n_per_w = n_total // NW
    mesh = plsc.VectorSubcoreMesh(
        core_axis_name="c", subcore_axis_name="s", num_cores=NC, num_subcores=NS
    )

    @functools.partial(
        pl.kernel,
        mesh=mesh,
        compiler_params=pltpu.CompilerParams(use_tc_tiling_on_sc=False),
        out_type=jax.ShapeDtypeStruct((n_total, EMB), jnp.float32),
        scratch_types=[
            pltpu.VMEM((CHUNK,), jnp.int32),
            pltpu.VMEM((CHUNK,), jnp.int32),
            pltpu.VMEM((CHUNK,), jnp.int32),
            pltpu.VMEM((CHUNK,), jnp.int32),
            pltpu.VMEM((SEQ, EMB), jnp.float32),
            pltpu.VMEM((CHUNK, EMB), jnp.float32),
            pltpu.VMEM((CHUNK, EMB), jnp.float32),
            pltpu.SemaphoreType.DMA,
            pltpu.SemaphoreType.DMA,
            pltpu.SemaphoreType.DMA,
            pltpu.SemaphoreType.DMA,
        ],
    )
    def body(idx_hbm, table_hbm, pos_hbm, out_hbm, i0, i1, i2, i3, pos_v,
             buf0, buf1, g0, g1, w0, w1):
        wid = lax.axis_index("s") * NC + lax.axis_index("c")
        base = wid * n_per_w
        idxs = (i0, i1, i2, i3)
        for c in range(N_CHUNKS):
            pltpu.sync_copy(idx_hbm.at[wid, c], idxs[c])
        pltpu.sync_copy(pos_hbm, pos_v)

        bufs = (buf0, buf1)
        gsems = (g0, g1)
        wsems = (w0, w1)

        def gather(c):
            b = c % 2
            return pltpu.async_copy(table_hbm.at[idxs[c]], bufs[b], gsems[b])

        def pos_add(b):
            buf = bufs[b]

            @plsc.parallel_loop(0, SEQ, unroll=2)
            def _(p):
                lo = pos_v[p, pl.ds(0, 16)]
                hi = pos_v[p, pl.ds(16, 16)]
                for s in range(CHUNK_SEQS):
                    r = s * SEQ + p
                    plsc.addupdate(buf.at[r, pl.ds(0, 16)], lo)
                    plsc.addupdate(buf.at[r, pl.ds(16, 16)], hi)

        def write(c):
            b = c % 2
            return pltpu.async_copy(
                bufs[b], out_hbm.at[pl.ds(base + c * CHUNK, CHUNK)], wsems[b]
            )

        pend_g = [gather(0), gather(1)]
        pend_w = []
        for c in range(N_CHUNKS):
            pend_g[c].wait()
            pos_add(c % 2)
            pend_w.append(write(c))
            if c + 2 < N_CHUNKS:
                pend_w[c].wait()
                pend_g.append(gather(c + 2))
        for c in range(N_CHUNKS - 2, N_CHUNKS):
            pend_w[c].wait()

    return body(idx_3d, table_rm, pos_table)


OUT_ROWS = 51200
RC = 200
R_CHUNKS = OUT_ROWS // NW // RC


def _retile_out(out_flat):
    mesh = plsc.VectorSubcoreMesh(
        core_axis_name="c", subcore_axis_name="s", num_cores=NC, num_subcores=NS
    )

    @functools.partial(
        pl.kernel,
        mesh=mesh,
        compiler_params=pltpu.CompilerParams(
            use_tc_tiling_on_sc=True, needs_layout_passes=False
        ),
        out_type=jax.ShapeDtypeStruct((OUT_ROWS, 128), jnp.float32),
        scratch_types=[
            pltpu.VMEM((RC * 128,), jnp.float32),
            pltpu.VMEM((RC * 128,), jnp.float32),
            pltpu.VMEM((RC, 128), jnp.float32),
            pltpu.VMEM((RC, 128), jnp.float32),
            pltpu.SemaphoreType.DMA,
            pltpu.SemaphoreType.DMA,
            pltpu.SemaphoreType.DMA,
            pltpu.SemaphoreType.DMA,
        ],
    )
    def body(in_hbm, out_hbm, f0, f1, t0, t1, gi0, gi1, wo0, wo1):
        wid = lax.axis_index("s") * NC + lax.axis_index("c")
        row0 = wid * (RC * R_CHUNKS)
        fls = (f0, f1)
        tws = (t0, t1)
        gsems = (gi0, gi1)
        wsems = (wo0, wo1)

        def fetch(c):
            b = c % 2
            return pltpu.async_copy(
                in_hbm.at[pl.ds((row0 + c * RC) * 128, RC * 128)],
                fls[b], gsems[b],
            )

        def conv(b):
            fl = fls[b]
            tw = tws[b]

            @plsc.parallel_loop(0, RC, unroll=4)
            def _(r):
                for g in range(8):
                    tw[r, pl.ds(g * LANES, LANES)] = (
                        fl[pl.ds(r * 128 + g * LANES, LANES)]
                    )

        def write(c):
            b = c % 2
            return pltpu.async_copy(
                tws[b], out_hbm.at[pl.ds(row0 + c * RC, RC)], wsems[b]
            )

        pend_g = [fetch(0), fetch(1)]
        pend_w = []
        for c in range(R_CHUNKS):
            pend_g[c].wait()
            if c >= 2:
                pend_w[c - 2].wait()
            conv(c % 2)
            pend_w.append(write(c))
            if c + 2 < R_CHUNKS:
                pend_g.append(fetch(c + 2))
        for c in range(R_CHUNKS - 2, R_CHUNKS):
            pend_w[c].wait()

    return body(out_flat)


@functools.partial(jax.jit, static_argnames=("n_total",))
def _embed_lookup(idx_flat, token_table, pos_table, n_total):
    table_t = token_table.T
    tail_flat = lax.slice(
        token_table, (N_FULL_TILES * TILE_W, 0), (VOCAB, EMB)
    ).reshape(TAIL * EMB)
    table_flat = _detile_table(table_t, tail_flat)
    table_rm = table_flat.reshape(VOCAB, EMB)
    idx_3d = idx_flat.reshape(NW, N_CHUNKS, CHUNK)
    out2 = _gather_add(idx_3d, table_rm, pos_table, n_total)
    return _retile_out(out2.reshape(n_total * EMB))


def kernel(inputs, token_table, pos_table):
    batch, seq_len = inputs.shape
    n_total = batch * seq_len
    out = _embed_lookup(inputs.reshape(n_total), token_table, pos_table, n_total)
    return out.reshape(batch, seq_len, EMB)


# speedup vs baseline: 1.0367x; 1.0367x over previous
import functools

import jax
import jax.numpy as jnp
from jax import lax
from jax.experimental import pallas as pl
from jax.experimental.pallas import tpu as pltpu
from jax.experimental.pallas import tpu_sc as plsc

SEQ = 200
EMB = 32
NC = 2
NS = 16
NW = NC * NS
LANES = 16

VOCAB = 1000000
TILE_W = 128
N_FULL_TILES = VOCAB // TILE_W
TAIL = VOCAB - N_FULL_TILES * TILE_W

CHUNK_SEQS = 8
CHUNK = CHUNK_SEQS * SEQ
N_CHUNKS = 4


BLK_TILES = 4
BLK_W = BLK_TILES * TILE_W
N_BLOCKS = N_FULL_TILES // BLK_TILES
SKEW = EMB + 1


def _detile_table(table_t, tail_flat):
    mesh = plsc.VectorSubcoreMesh(
        core_axis_name="c", subcore_axis_name="s", num_cores=NC, num_subcores=NS
    )

    @functools.partial(
        pl.kernel,
        mesh=mesh,
        compiler_params=pltpu.CompilerParams(
            use_tc_tiling_on_sc=True, needs_layout_passes=False
        ),
        out_type=jax.ShapeDtypeStruct((VOCAB * EMB,), jnp.float32),
        scratch_types=[
            pltpu.VMEM((EMB, BLK_W), jnp.float32),
            pltpu.VMEM((EMB, BLK_W), jnp.float32),
            pltpu.VMEM((BLK_W * SKEW,), jnp.float32),
            pltpu.VMEM((BLK_W * EMB,), jnp.float32),
            pltpu.VMEM((BLK_W * EMB,), jnp.float32),
            pltpu.VMEM((TAIL * EMB,), jnp.float32),
            pltpu.SemaphoreType.DMA,
            pltpu.SemaphoreType.DMA,
            pltpu.SemaphoreType.DMA,
            pltpu.SemaphoreType.DMA,
        ],
    )
    def body(tt_hbm, tail_hbm, out_hbm, in0, in1, skew_v, st0, st1, tailv,
             gi0, gi1, wo0, wo1):
        wid = lax.axis_index("s") * NC + lax.axis_index("c")
        ins = (in0, in1)
        sts = (st0, st1)
        gsems = (gi0, gi1)
        wsems = (wo0, wo1)

        n_mine = N_BLOCKS // NW + jnp.where(wid < N_BLOCKS % NW, 1, 0)

        lane = lax.iota(jnp.int32, LANES)
        scat_base = lane * SKEW

        def fetch(k, b):
            v0 = (wid + k * NW) * BLK_W
            for t in range(EMB // 8):
                pltpu.async_copy(
                    tt_hbm.at[pl.ds(8 * t, 8), pl.ds(v0, BLK_W)],
                    ins[b].at[pl.ds(8 * t, 8), :],
                    gsems[b],
                )

        def transpose(b):
            buf = ins[b]
            stg = sts[b]

            @plsc.parallel_loop(0, EMB, unroll=4)
            def _(d):
                for grp in range(BLK_W // LANES):
                    x = buf[d, pl.ds(grp * LANES, LANES)]
                    idx = scat_base + (grp * LANES * SKEW + d)
                    plsc.store_scatter(skew_v, [idx], x)

            @plsc.parallel_loop(0, BLK_W, unroll=8)
            def _(r):
                stg[pl.ds(r * EMB, LANES)] = skew_v[pl.ds(r * SKEW, LANES)]
                stg[pl.ds(r * EMB + LANES, LANES)] = (
                    skew_v[pl.ds(r * SKEW + LANES, LANES)]
                )

        def write(k, b):
            v0 = (wid + k * NW) * BLK_W
            return pltpu.async_copy(
                sts[b], out_hbm.at[pl.ds(v0 * EMB, BLK_W * EMB)], wsems[b]
            )

        fetch(0, 0)

        def pair_body(kp, carry):
            for b in range(2):
                k = kp * 2 + b

                @pl.when(k < n_mine)
                def _():
                    @pl.when(k + 1 < n_mine)
                    def _():
                        fetch(k + 1, 1 - b)

                    @pl.when(k >= 2)
                    def _():
                        pltpu.make_async_copy(
                            sts[b], out_hbm.at[pl.ds(0, BLK_W * EMB)],
                            wsems[b],
                        ).wait()

                    pltpu.make_async_copy(
                        tt_hbm.at[:, pl.ds(0, BLK_W)], ins[b], gsems[b]
                    ).wait()
                    transpose(b)
                    write(k, b)

            return carry

        lax.fori_loop(0, (n_mine + 1) // 2, pair_body, 0)

        for b in range(2):
            pltpu.make_async_copy(
                sts[b], out_hbm.at[pl.ds(0, BLK_W * EMB)], wsems[b]
            ).wait()

        @pl.when(wid == 0)
        def _():
            pltpu.sync_copy(tail_hbm, tailv)
            pltpu.sync_copy(
                tailv,
                out_hbm.at[pl.ds(N_FULL_TILES * TILE_W * EMB, TAIL * EMB)],
            )

    return body(table_t, tail_flat)


def _gather_add(idx_3d, table_rm, pos_table, n_total):
    n_per_w = n_total // NW
    mesh = plsc.VectorSubcoreMesh(
        core_axis_name="c", subcore_axis_name="s", num_cores=NC, num_subcores=NS
    )

    @functools.partial(
        pl.kernel,
        mesh=mesh,
        compiler_params=pltpu.CompilerParams(use_tc_tiling_on_sc=False),
        out_type=jax.ShapeDtypeStruct((n_total, EMB), jnp.float32),
        scratch_types=[
            pltpu.VMEM((CHUNK,), jnp.int32),
            pltpu.VMEM((CHUNK,), jnp.int32),
            pltpu.VMEM((CHUNK,), jnp.int32),
            pltpu.VMEM((CHUNK,), jnp.int32),
            pltpu.VMEM((SEQ, EMB), jnp.float32),
            pltpu.VMEM((CHUNK, EMB), jnp.float32),
            pltpu.VMEM((CHUNK, EMB), jnp.float32),
            pltpu.SemaphoreType.DMA,
            pltpu.SemaphoreType.DMA,
            pltpu.SemaphoreType.DMA,
            pltpu.SemaphoreType.DMA,
        ],
    )
    def body(idx_hbm, table_hbm, pos_hbm, out_hbm, i0, i1, i2, i3, pos_v,
             buf0, buf1, g0, g1, w0, w1):
        wid = lax.axis_index("s") * NC + lax.axis_index("c")
        base = wid * n_per_w
        idxs = (i0, i1, i2, i3)
        for c in range(N_CHUNKS):
            pltpu.sync_copy(idx_hbm.at[wid, c], idxs[c])
        pltpu.sync_copy(pos_hbm, pos_v)

        bufs = (buf0, buf1)
        gsems = (g0, g1)
        wsems = (w0, w1)

        def gather(c):
            b = c % 2
            return pltpu.async_copy(table_hbm.at[idxs[c]], bufs[b], gsems[b])

        def pos_add(b):
            buf = bufs[b]

            @plsc.parallel_loop(0, SEQ, unroll=2)
            def _(p):
                lo = pos_v[p, pl.ds(0, 16)]
                hi = pos_v[p, pl.ds(16, 16)]
                for s in range(CHUNK_SEQS):
                    r = s * SEQ + p
                    plsc.addupdate(buf.at[r, pl.ds(0, 16)], lo)
                    plsc.addupdate(buf.at[r, pl.ds(16, 16)], hi)

        def write(c):
            b = c % 2
            return pltpu.async_copy(
                bufs[b], out_hbm.at[pl.ds(base + c * CHUNK, CHUNK)], wsems[b]
            )

        pend_g = [gather(0), gather(1)]
        pend_w = []
        for c in range(N_CHUNKS):
            pend_g[c].wait()
            pos_add(c % 2)
            pend_w.append(write(c))
            if c + 2 < N_CHUNKS:
                pend_w[c].wait()
                pend_g.append(gather(c + 2))
        for c in range(N_CHUNKS - 2, N_CHUNKS):
            pend_w[c].wait()

    return body(idx_3d, table_rm, pos_table)


@functools.partial(jax.jit, static_argnames=("n_total",))
def _embed_lookup(idx_flat, token_table, pos_table, n_total):
    table_t = token_table.T
    tail_flat = lax.slice(
        token_table, (N_FULL_TILES * TILE_W, 0), (VOCAB, EMB)
    ).reshape(TAIL * EMB)
    table_flat = _detile_table(table_t, tail_flat)
    table_rm = table_flat.reshape(VOCAB, EMB)
    idx_3d = idx_flat.reshape(NW, N_CHUNKS, CHUNK)
    return _gather_add(idx_3d, table_rm, pos_table, n_total)


def kernel(inputs, token_table, pos_table):
    batch, seq_len = inputs.shape
    n_total = batch * seq_len
    out = _embed_lookup(inputs.reshape(n_total), token_table, pos_table, n_total)
    return out.reshape(batch, seq_len, EMB)
